# baseline (speedup 1.0000x reference)
import functools

import jax
import jax.numpy as jnp
from jax import lax
from jax.experimental import pallas as pl
from jax.experimental.pallas import tpu as pltpu
from jax.experimental.pallas import tpu_sc as plsc

NUM_USER = 25000
NUM_ITEM = 25000
N_NODES = NUM_USER + NUM_ITEM
N_EDGES = 800000
DIM_FEAT = 128
DIM = 64
HALF = 32

BR = 5000
NB = NUM_USER // BR


def _dense_body(pref_ref, feat_ref, w_ref, b_ref, cw_ref, out_ref):
    g = pl.program_id(0)
    t = lax.dot_general(feat_ref[...], w_ref[...],
                        (((1,), (1,)), ((), ())),
                        preferred_element_type=jnp.float32) + b_ref[...]
    x = jnp.where(g == 0, pref_ref[...], t)
    norm = jnp.sqrt(jnp.sum(x * x, axis=1, keepdims=True))
    x = x / jnp.maximum(norm, 1e-12)
    y = lax.dot_general(x, cw_ref[...], (((1,), (0,)), ((), ())),
                        preferred_element_type=jnp.float32)
    out_ref[0, :, :] = y[:, :HALF]
    out_ref[1, :, :] = y[:, HALF:]


_dense = pl.pallas_call(
    _dense_body,
    grid=(2, NB),
    in_specs=[
        pl.BlockSpec((BR, DIM), lambda g, j: (jnp.where(g == 0, j, 0), 0)),
        pl.BlockSpec((BR, DIM_FEAT), lambda g, j: (jnp.where(g == 0, 0, j), 0)),
        pl.BlockSpec((DIM, DIM_FEAT), lambda g, j: (0, 0)),
        pl.BlockSpec((1, DIM), lambda g, j: (0, 0)),
        pl.BlockSpec((DIM, DIM), lambda g, j: (0, 0)),
    ],
    out_specs=pl.BlockSpec((2, BR, HALF), lambda g, j: (0, g * NB + j, 0)),
    out_shape=jax.ShapeDtypeStruct((2, N_NODES, HALF), jnp.float32),
)

NC = 2
NS = 16
SUB = 128
SUBS = 2
CH = SUBS * SUB
EROWS = N_EDGES // SUB
ERPT = 390
N_CH = ERPT // SUBS
NSLOT = N_CH + 3
EXTRA_BASE = NS * ERPT
RPT = 3136
ACC_ROWS = NS * RPT
DRAIN = 224
NDRAIN = RPT // DRAIN
LAST_R0 = 49952


def _sc_body(xw_hbm, ed_hbm, out_hbm,
             src_v, dst_v, rows_v, acc_sh, gsem, ssem, isem):
    cid = lax.axis_index("c")
    sid = lax.axis_index("s")
    z16 = jnp.zeros((16,), jnp.float32)
    xw_c = xw_hbm.at[cid]

    def _idx_load(c, r, wait):
        base = (sid * ERPT + c * SUBS) * SUB
        for ref, e in ((src_v[r], 0), (dst_v[r], 1)):
            for j in range(SUBS):
                cp = pltpu.make_async_copy(
                    ed_hbm.at[e, pl.ds(base + j * SUB, SUB)],
                    ref.at[j], isem[r])
                cp.wait() if wait else cp.start()

    def _gather(r, wait):
        for j in range(SUBS):
            cp = pltpu.make_async_copy(
                xw_c.at[src_v[r].at[j]],
                rows_v[r].at[pl.ds(j * SUB, SUB)], gsem[r])
            cp.wait() if wait else cp.start()

    def _scatter(r, wait):
        for j in range(SUBS):
            cp = pltpu.make_async_copy(
                rows_v[r].at[pl.ds(j * SUB, SUB)],
                acc_sh.at[dst_v[r].at[j]], ssem[r])
            cp.wait() if wait else cp.start(add=True)

    def _zrow(i, _):
        rows_v[0][i, 0:16] = z16
        rows_v[0][i, 16:32] = z16
        return 0

    lax.fori_loop(0, DRAIN, _zrow, 0)
    abase = sid * RPT

    def _zchunk(k, _):
        pltpu.sync_copy(rows_v[0].at[pl.ds(0, DRAIN)],
                        acc_sh.at[pl.ds(abase + k * DRAIN, DRAIN)])
        return 0

    lax.fori_loop(0, NDRAIN, _zchunk, 0)
    plsc.subcore_barrier()

    _idx_load(0, 0, wait=False)
    _idx_load(0, 0, wait=True)
    _gather(0, wait=False)

    def _edge_iter(o, _):
        for k in range(3):
            c = 3 * o + k

            @pl.when((c >= 2) & (c <= N_CH + 1))
            def _():
                _scatter((k + 1) % 3, wait=True)

            @pl.when(c + 1 < N_CH)
            def _():
                _idx_load(c + 1, (k + 1) % 3, wait=False)

            @pl.when(c < N_CH)
            def _():
                _gather(k, wait=True)
                _scatter(k, wait=False)

            @pl.when(c + 1 < N_CH)
            def _():
                _idx_load(c + 1, (k + 1) % 3, wait=True)
                _gather((k + 1) % 3, wait=False)
        return 0

    lax.fori_loop(0, NSLOT // 3, _edge_iter, 0)

    @pl.when(sid < EROWS - EXTRA_BASE)
    def _():
        for ref, e in ((src_v[0], 0), (dst_v[0], 1)):
            pltpu.sync_copy(
                ed_hbm.at[e, pl.ds((EXTRA_BASE + sid) * SUB, SUB)],
                ref.at[0])
        pltpu.make_async_copy(xw_c.at[src_v[0].at[0]],
                              rows_v[0].at[pl.ds(0, SUB)], gsem[0]).start()
        pltpu.make_async_copy(xw_c.at[src_v[0].at[0]],
                              rows_v[0].at[pl.ds(0, SUB)], gsem[0]).wait()
        pltpu.make_async_copy(rows_v[0].at[pl.ds(0, SUB)],
                              acc_sh.at[dst_v[0].at[0]], ssem[0]).start(add=True)
        pltpu.make_async_copy(rows_v[0].at[pl.ds(0, SUB)],
                              acc_sh.at[dst_v[0].at[0]], ssem[0]).wait()

    plsc.subcore_barrier()

    def _drain(k, _):
        r0 = abase + k * DRAIN
        pltpu.sync_copy(acc_sh.at[pl.ds(r0, DRAIN)],
                        rows_v[0].at[pl.ds(0, DRAIN)])

        def _lr(i, _):
            v0 = rows_v[0][i, 0:16]
            rows_v[0][i, 0:16] = jnp.maximum(v0, v0 * 0.01)
            v1 = rows_v[0][i, 16:32]
            rows_v[0][i, 16:32] = jnp.maximum(v1, v1 * 0.01)
            return 0

        lax.fori_loop(0, DRAIN, _lr, 0)

        @pl.when(r0 <= N_NODES - DRAIN)
        def _():
            pltpu.sync_copy(
                rows_v[0].at[pl.ds(0, DRAIN)],
                out_hbm.at[pl.ds(r0, DRAIN), pl.ds(cid * HALF, HALF)])

        @pl.when(r0 == LAST_R0)
        def _():
            pltpu.sync_copy(
                rows_v[0].at[pl.ds(0, N_NODES - LAST_R0)],
                out_hbm.at[pl.ds(LAST_R0, N_NODES - LAST_R0),
                           pl.ds(cid * HALF, HALF)])
        return 0

    lax.fori_loop(0, NDRAIN, _drain, 0)


@functools.cache
def _make_sc_agg():
    mesh = plsc.VectorSubcoreMesh(core_axis_name="c", subcore_axis_name="s",
                                  num_cores=NC, num_subcores=NS)
    return pl.kernel(
        _sc_body,
        out_type=jax.ShapeDtypeStruct((N_NODES, DIM), jnp.float32),
        mesh=mesh,
        scratch_types=[
            [pltpu.VMEM((SUBS, SUB), jnp.int32) for _ in range(3)],
            [pltpu.VMEM((SUBS, SUB), jnp.int32) for _ in range(3)],
            [pltpu.VMEM((CH, HALF), jnp.float32) for _ in range(3)],
            pltpu.VMEM_SHARED((ACC_ROWS, HALF), jnp.float32),
            [pltpu.SemaphoreType.DMA for _ in range(3)],
            [pltpu.SemaphoreType.DMA for _ in range(3)],
            [pltpu.SemaphoreType.DMA for _ in range(3)],
        ],
        compiler_params=pltpu.CompilerParams(use_tc_tiling_on_sc=False),
    )


def kernel(features, edge_index, id_embedding, preference, MLP_W, MLP_b, conv_W):
    del id_embedding
    xw = _dense(preference, features, MLP_W, MLP_b.reshape(1, DIM), conv_W)
    return _make_sc_agg()(xw, edge_index.astype(jnp.int32))


# speedup vs baseline: 1.0494x; 1.0494x over previous
import functools

import jax
import jax.numpy as jnp
from jax import lax
from jax.experimental import pallas as pl
from jax.experimental.pallas import tpu as pltpu
from jax.experimental.pallas import tpu_sc as plsc

NUM_USER = 25000
NUM_ITEM = 25000
N_NODES = NUM_USER + NUM_ITEM
N_EDGES = 800000
DIM_FEAT = 128
DIM = 64
HALF = 32

BR = 1000
NB = NUM_USER // BR


def _dense_body(pref_ref, feat_ref, w_ref, b_ref, cw_ref, out_ref):
    g = pl.program_id(0)
    t = lax.dot_general(feat_ref[...], w_ref[...],
                        (((1,), (1,)), ((), ())),
                        preferred_element_type=jnp.float32) + b_ref[...]
    x = jnp.where(g == 0, pref_ref[...], t)
    norm = jnp.sqrt(jnp.sum(x * x, axis=1, keepdims=True))
    x = x / jnp.maximum(norm, 1e-12)
    y = lax.dot_general(x, cw_ref[...], (((1,), (0,)), ((), ())),
                        preferred_element_type=jnp.float32)
    out_ref[0, :, :] = y[:, :HALF]
    out_ref[1, :, :] = y[:, HALF:]


_dense = pl.pallas_call(
    _dense_body,
    grid=(2, NB),
    in_specs=[
        pl.BlockSpec((BR, DIM), lambda g, j: (jnp.where(g == 0, j, 0), 0)),
        pl.BlockSpec((BR, DIM_FEAT), lambda g, j: (jnp.where(g == 0, 0, j), 0)),
        pl.BlockSpec((DIM, DIM_FEAT), lambda g, j: (0, 0)),
        pl.BlockSpec((1, DIM), lambda g, j: (0, 0)),
        pl.BlockSpec((DIM, DIM), lambda g, j: (0, 0)),
    ],
    out_specs=pl.BlockSpec((2, BR, HALF), lambda g, j: (0, g * NB + j, 0)),
    out_shape=jax.ShapeDtypeStruct((2, N_NODES, HALF), jnp.float32),
)

NC = 2
NS = 16
SUB = 64
SUBS = 4
CH = SUBS * SUB
EROWS = N_EDGES // SUB
ERPT = 780
N_CH = ERPT // SUBS
NSLOT = N_CH + 3
EXTRA_BASE = NS * ERPT
RPT = 3136
ACC_ROWS = NS * RPT
DRAIN = 224
NDRAIN = RPT // DRAIN
LAST_R0 = 49952


def _sc_body(xw_hbm, ed_hbm, out_hbm,
             src_v, dst_v, rows_v, acc_sh, gsem, ssem, isem):
    cid = lax.axis_index("c")
    sid = lax.axis_index("s")
    z16 = jnp.zeros((16,), jnp.float32)
    xw_c = xw_hbm.at[cid]

    def _idx_load(c, r, wait):
        base = (sid * ERPT + c * SUBS) * SUB
        for ref, e in ((src_v[r], 0), (dst_v[r], 1)):
            for j in range(SUBS):
                cp = pltpu.make_async_copy(
                    ed_hbm.at[e, pl.ds(base + j * SUB, SUB)],
                    ref.at[j], isem[r])
                cp.wait() if wait else cp.start()

    def _gather(r, wait):
        for j in range(SUBS):
            cp = pltpu.make_async_copy(
                xw_c.at[src_v[r].at[j]],
                rows_v[r].at[pl.ds(j * SUB, SUB)], gsem[r])
            cp.wait() if wait else cp.start()

    def _scatter(r, wait):
        for j in range(SUBS):
            cp = pltpu.make_async_copy(
                rows_v[r].at[pl.ds(j * SUB, SUB)],
                acc_sh.at[dst_v[r].at[j]], ssem[r])
            cp.wait() if wait else cp.start(add=True)

    def _zrow(i, _):
        rows_v[0][i, 0:16] = z16
        rows_v[0][i, 16:32] = z16
        return 0

    lax.fori_loop(0, DRAIN, _zrow, 0)
    abase = sid * RPT

    def _zchunk(k, _):
        pltpu.sync_copy(rows_v[0].at[pl.ds(0, DRAIN)],
                        acc_sh.at[pl.ds(abase + k * DRAIN, DRAIN)])
        return 0

    lax.fori_loop(0, NDRAIN, _zchunk, 0)
    plsc.subcore_barrier()

    _idx_load(0, 0, wait=False)
    _idx_load(0, 0, wait=True)
    _gather(0, wait=False)

    def _edge_iter(o, _):
        for k in range(3):
            c = 3 * o + k

            @pl.when((c >= 2) & (c <= N_CH + 1))
            def _():
                _scatter((k + 1) % 3, wait=True)

            @pl.when(c + 1 < N_CH)
            def _():
                _idx_load(c + 1, (k + 1) % 3, wait=False)

            @pl.when(c < N_CH)
            def _():
                _gather(k, wait=True)
                _scatter(k, wait=False)

            @pl.when(c + 1 < N_CH)
            def _():
                _idx_load(c + 1, (k + 1) % 3, wait=True)
                _gather((k + 1) % 3, wait=False)
        return 0

    lax.fori_loop(0, NSLOT // 3, _edge_iter, 0)

    def _extra(row):
        for ref, e in ((src_v[0], 0), (dst_v[0], 1)):
            pltpu.sync_copy(ed_hbm.at[e, pl.ds(row * SUB, SUB)], ref.at[0])
        pltpu.make_async_copy(xw_c.at[src_v[0].at[0]],
                              rows_v[0].at[pl.ds(0, SUB)], gsem[0]).start()
        pltpu.make_async_copy(xw_c.at[src_v[0].at[0]],
                              rows_v[0].at[pl.ds(0, SUB)], gsem[0]).wait()
        pltpu.make_async_copy(rows_v[0].at[pl.ds(0, SUB)],
                              acc_sh.at[dst_v[0].at[0]], ssem[0]).start(add=True)
        pltpu.make_async_copy(rows_v[0].at[pl.ds(0, SUB)],
                              acc_sh.at[dst_v[0].at[0]], ssem[0]).wait()

    _extra(EXTRA_BASE + sid)

    @pl.when(sid < EROWS - EXTRA_BASE - NS)
    def _():
        _extra(EXTRA_BASE + NS + sid)

    plsc.subcore_barrier()

    def _drain(k, _):
        r0 = abase + k * DRAIN
        pltpu.sync_copy(acc_sh.at[pl.ds(r0, DRAIN)],
                        rows_v[0].at[pl.ds(0, DRAIN)])

        def _lr(i, _):
            v0 = rows_v[0][i, 0:16]
            rows_v[0][i, 0:16] = jnp.maximum(v0, v0 * 0.01)
            v1 = rows_v[0][i, 16:32]
            rows_v[0][i, 16:32] = jnp.maximum(v1, v1 * 0.01)
            return 0

        lax.fori_loop(0, DRAIN, _lr, 0)

        @pl.when(r0 <= N_NODES - DRAIN)
        def _():
            pltpu.sync_copy(
                rows_v[0].at[pl.ds(0, DRAIN)],
                out_hbm.at[pl.ds(r0, DRAIN), pl.ds(cid * HALF, HALF)])

        @pl.when(r0 == LAST_R0)
        def _():
            pltpu.sync_copy(
                rows_v[0].at[pl.ds(0, N_NODES - LAST_R0)],
                out_hbm.at[pl.ds(LAST_R0, N_NODES - LAST_R0),
                           pl.ds(cid * HALF, HALF)])
        return 0

    lax.fori_loop(0, NDRAIN, _drain, 0)


@functools.cache
def _make_sc_agg():
    mesh = plsc.VectorSubcoreMesh(core_axis_name="c", subcore_axis_name="s",
                                  num_cores=NC, num_subcores=NS)
    return pl.kernel(
        _sc_body,
        out_type=jax.ShapeDtypeStruct((N_NODES, DIM), jnp.float32),
        mesh=mesh,
        scratch_types=[
            [pltpu.VMEM((SUBS, SUB), jnp.int32) for _ in range(3)],
            [pltpu.VMEM((SUBS, SUB), jnp.int32) for _ in range(3)],
            [pltpu.VMEM((CH, HALF), jnp.float32) for _ in range(3)],
            pltpu.VMEM_SHARED((ACC_ROWS, HALF), jnp.float32),
            [pltpu.SemaphoreType.DMA for _ in range(3)],
            [pltpu.SemaphoreType.DMA for _ in range(3)],
            [pltpu.SemaphoreType.DMA for _ in range(3)],
        ],
        compiler_params=pltpu.CompilerParams(use_tc_tiling_on_sc=False),
    )


def kernel(features, edge_index, id_embedding, preference, MLP_W, MLP_b, conv_W):
    del id_embedding
    xw = _dense(preference, features, MLP_W, MLP_b.reshape(1, DIM), conv_W)
    return _make_sc_agg()(xw, edge_index.astype(jnp.int32))


# speedup vs baseline: 1.0641x; 1.0141x over previous
import functools

import jax
import jax.numpy as jnp
from jax import lax
from jax.experimental import pallas as pl
from jax.experimental.pallas import tpu as pltpu
from jax.experimental.pallas import tpu_sc as plsc

NUM_USER = 25000
NUM_ITEM = 25000
N_NODES = NUM_USER + NUM_ITEM
N_EDGES = 800000
DIM_FEAT = 128
DIM = 64
HALF = 32

BR = 1000
NB = NUM_USER // BR


def _dense_body(pref_ref, feat_ref, w_ref, b_ref, cw_ref, out_ref):
    g = pl.program_id(0)
    t = lax.dot_general(feat_ref[...], w_ref[...],
                        (((1,), (1,)), ((), ())),
                        preferred_element_type=jnp.float32) + b_ref[...]
    x = jnp.where(g == 0, pref_ref[...], t)
    norm = jnp.sqrt(jnp.sum(x * x, axis=1, keepdims=True))
    x = x / jnp.maximum(norm, 1e-12)
    y = lax.dot_general(x, cw_ref[...], (((1,), (0,)), ((), ())),
                        preferred_element_type=jnp.float32)
    out_ref[0, :, :] = y[:, :HALF]
    out_ref[1, :, :] = y[:, HALF:]


_dense = pl.pallas_call(
    _dense_body,
    grid=(2, NB),
    in_specs=[
        pl.BlockSpec((BR, DIM), lambda g, j: (jnp.where(g == 0, j, 0), 0)),
        pl.BlockSpec((BR, DIM_FEAT), lambda g, j: (jnp.where(g == 0, 0, j), 0)),
        pl.BlockSpec((DIM, DIM_FEAT), lambda g, j: (0, 0)),
        pl.BlockSpec((1, DIM), lambda g, j: (0, 0)),
        pl.BlockSpec((DIM, DIM), lambda g, j: (0, 0)),
    ],
    out_specs=pl.BlockSpec((2, BR, HALF), lambda g, j: (0, g * NB + j, 0)),
    out_shape=jax.ShapeDtypeStruct((2, N_NODES, HALF), jnp.float32),
)

NC = 2
NS = 16
SUB = 64
SUBS = 4
CH = SUBS * SUB
EROWS = N_EDGES // SUB
ERPT = 780
N_CH = ERPT // SUBS
NSLOT = N_CH + 3
EXTRA_BASE = NS * ERPT
RPT = 3136
ACC_ROWS = NS * RPT
DRAIN = 224
NDRAIN = RPT // DRAIN
LAST_R0 = 49952


def _sc_body(xw_hbm, ed_hbm, out_hbm,
             src_v, dst_v, rows_v, acc_sh, gsem, ssem, isem):
    cid = lax.axis_index("c")
    sid = lax.axis_index("s")
    z16 = jnp.zeros((16,), jnp.float32)
    xw_c = xw_hbm.at[cid]

    def _idx_load(c, r, wait):
        base = (sid * ERPT + c * SUBS) * SUB
        for ref, e in ((src_v[r], 0), (dst_v[r], 1)):
            for j in range(SUBS):
                cp = pltpu.make_async_copy(
                    ed_hbm.at[e, pl.ds(base + j * SUB, SUB)],
                    ref.at[j], isem[r])
                cp.wait() if wait else cp.start()

    def _gather(r, wait):
        for j in range(SUBS):
            cp = pltpu.make_async_copy(
                xw_c.at[src_v[r].at[j]],
                rows_v[r].at[pl.ds(j * SUB, SUB)], gsem[r])
            cp.wait() if wait else cp.start()

    def _scatter(r, wait):
        for j in range(SUBS):
            cp = pltpu.make_async_copy(
                rows_v[r].at[pl.ds(j * SUB, SUB)],
                acc_sh.at[dst_v[r].at[j]], ssem[r])
            cp.wait() if wait else cp.start(add=True)

    def _zrow(i, _):
        rows_v[2][i, 0:16] = z16
        rows_v[2][i, 16:32] = z16
        return 0

    lax.fori_loop(0, DRAIN, _zrow, 0)
    abase = sid * RPT

    def _zchunk(wait):
        def body(k, _):
            cp = pltpu.make_async_copy(
                rows_v[2].at[pl.ds(0, DRAIN)],
                acc_sh.at[pl.ds(abase + k * DRAIN, DRAIN)], ssem[2])
            cp.wait() if wait else cp.start()
            return 0
        return body

    lax.fori_loop(0, NDRAIN, _zchunk(False), 0)
    lax.fori_loop(0, NDRAIN, _zchunk(True), 0)
    plsc.subcore_barrier()

    _idx_load(0, 0, wait=False)
    _idx_load(0, 0, wait=True)
    _gather(0, wait=False)

    def _edge_iter(o, _):
        for k in range(3):
            c = 3 * o + k

            @pl.when((c >= 2) & (c <= N_CH + 1))
            def _():
                _scatter((k + 1) % 3, wait=True)

            @pl.when(c + 1 < N_CH)
            def _():
                _idx_load(c + 1, (k + 1) % 3, wait=False)

            @pl.when(c < N_CH)
            def _():
                _gather(k, wait=True)
                _scatter(k, wait=False)

            @pl.when(c + 1 < N_CH)
            def _():
                _idx_load(c + 1, (k + 1) % 3, wait=True)
                _gather((k + 1) % 3, wait=False)
        return 0

    lax.fori_loop(0, NSLOT // 3, _edge_iter, 0)

    def _extra(row):
        for ref, e in ((src_v[0], 0), (dst_v[0], 1)):
            pltpu.sync_copy(ed_hbm.at[e, pl.ds(row * SUB, SUB)], ref.at[0])
        pltpu.make_async_copy(xw_c.at[src_v[0].at[0]],
                              rows_v[0].at[pl.ds(0, SUB)], gsem[0]).start()
        pltpu.make_async_copy(xw_c.at[src_v[0].at[0]],
                              rows_v[0].at[pl.ds(0, SUB)], gsem[0]).wait()
        pltpu.make_async_copy(rows_v[0].at[pl.ds(0, SUB)],
                              acc_sh.at[dst_v[0].at[0]], ssem[0]).start(add=True)
        pltpu.make_async_copy(rows_v[0].at[pl.ds(0, SUB)],
                              acc_sh.at[dst_v[0].at[0]], ssem[0]).wait()

    _extra(EXTRA_BASE + sid)

    @pl.when(sid < EROWS - EXTRA_BASE - NS)
    def _():
        _extra(EXTRA_BASE + NS + sid)

    plsc.subcore_barrier()

    def _d_in(k, r, wait):
        cp = pltpu.make_async_copy(
            acc_sh.at[pl.ds(abase + k * DRAIN, DRAIN)],
            rows_v[r].at[pl.ds(0, DRAIN)], gsem[r])
        cp.wait() if wait else cp.start()

    def _d_out(k, r, wait):
        r0 = abase + k * DRAIN

        @pl.when(r0 <= N_NODES - DRAIN)
        def _():
            cp = pltpu.make_async_copy(
                rows_v[r].at[pl.ds(0, DRAIN)],
                out_hbm.at[pl.ds(r0, DRAIN), pl.ds(cid * HALF, HALF)],
                ssem[r])
            cp.wait() if wait else cp.start()

        @pl.when(r0 == LAST_R0)
        def _():
            cp = pltpu.make_async_copy(
                rows_v[r].at[pl.ds(0, N_NODES - LAST_R0)],
                out_hbm.at[pl.ds(LAST_R0, N_NODES - LAST_R0),
                           pl.ds(cid * HALF, HALF)], ssem[r])
            cp.wait() if wait else cp.start()

    _d_in(0, 0, wait=False)

    def _drain(t, _):
        for k2 in range(2):
            k = 2 * t + k2
            _d_in(k, k2, wait=True)

            @pl.when(k >= 1)
            def _():
                _d_out(k - 1, 1 - k2, wait=True)

            @pl.when(k + 1 < NDRAIN)
            def _():
                _d_in(k + 1, 1 - k2, wait=False)

            def _lr(i, _):
                v0 = rows_v[k2][i, 0:16]
                rows_v[k2][i, 0:16] = jnp.maximum(v0, v0 * 0.01)
                v1 = rows_v[k2][i, 16:32]
                rows_v[k2][i, 16:32] = jnp.maximum(v1, v1 * 0.01)
                return 0

            lax.fori_loop(0, DRAIN, _lr, 0)
            _d_out(k, k2, wait=False)
        return 0

    lax.fori_loop(0, NDRAIN // 2, _drain, 0)
    _d_out(NDRAIN - 1, 1, wait=True)


@functools.cache
def _make_sc_agg():
    mesh = plsc.VectorSubcoreMesh(core_axis_name="c", subcore_axis_name="s",
                                  num_cores=NC, num_subcores=NS)
    return pl.kernel(
        _sc_body,
        out_type=jax.ShapeDtypeStruct((N_NODES, DIM), jnp.float32),
        mesh=mesh,
        scratch_types=[
            [pltpu.VMEM((SUBS, SUB), jnp.int32) for _ in range(3)],
            [pltpu.VMEM((SUBS, SUB), jnp.int32) for _ in range(3)],
            [pltpu.VMEM((CH, HALF), jnp.float32) for _ in range(3)],
            pltpu.VMEM_SHARED((ACC_ROWS, HALF), jnp.float32),
            [pltpu.SemaphoreType.DMA for _ in range(3)],
            [pltpu.SemaphoreType.DMA for _ in range(3)],
            [pltpu.SemaphoreType.DMA for _ in range(3)],
        ],
        compiler_params=pltpu.CompilerParams(use_tc_tiling_on_sc=False),
    )


def kernel(features, edge_index, id_embedding, preference, MLP_W, MLP_b, conv_W):
    del id_embedding
    xw = _dense(preference, features, MLP_W, MLP_b.reshape(1, DIM), conv_W)
    return _make_sc_agg()(xw, edge_index.astype(jnp.int32))
